# baseline (speedup 1.0000x reference)
import functools

import jax
import jax.numpy as jnp
from jax import lax
from jax.experimental import pallas as pl
from jax.experimental.pallas import tpu as pltpu
from jax.experimental.pallas import tpu_sc as plsc

_H = 32
_W = 32
_D = 768
_NC = 2
_NS = 16
_NW = _NC * _NS
_DP = _D // _NW
_L = 16

_mesh = plsc.VectorSubcoreMesh(core_axis_name="c", subcore_axis_name="s")


@functools.partial(
    pl.kernel,
    mesh=_mesh,
    out_type=jax.ShapeDtypeStruct((_D, _H, _W), jnp.float32),
    compiler_params=pltpu.CompilerParams(
        use_tc_tiling_on_sc=False, needs_layout_passes=False
    ),
    scratch_types=[
        pltpu.VMEM((_H, _D), jnp.float32),
        pltpu.VMEM((_W, _D), jnp.float32),
        pltpu.VMEM((_DP, _H, _W), jnp.float32),
    ],
)
def _pos2d(row_hbm, col_hbm, out_hbm, row_v, col_v, buf_v):
    wid = lax.axis_index("s") * _NC + lax.axis_index("c")
    d0 = wid * _DP
    pltpu.sync_copy(row_hbm, row_v)
    pltpu.sync_copy(col_hbm, col_v)

    iota = lax.iota(jnp.int32, _L)

    def body(dd, carry):
        dsplat = jnp.full((_L,), d0 + dd, jnp.int32)
        c0 = plsc.load_gather(col_v, [iota, dsplat])
        c1 = plsc.load_gather(col_v, [iota + _L, dsplat])
        r0 = plsc.load_gather(row_v, [iota, dsplat])
        r1 = plsc.load_gather(row_v, [iota + _L, dsplat])
        for j, rvec in ((0, r0), (1, r1)):
            for hh in range(_L):
                h = j * _L + hh
                rv = jnp.full((_L,), rvec[hh])
                buf_v[dd, h, pl.ds(0, _L)] = rv + c0
                buf_v[dd, h, pl.ds(_L, _L)] = rv + c1
        return carry

    lax.fori_loop(0, _DP, body, 0)
    pltpu.sync_copy(buf_v, out_hbm.at[pl.ds(d0, _DP)])


def kernel(row_embed, col_embed):
    return _pos2d(row_embed, col_embed)


# speedup vs baseline: 1.2976x; 1.2976x over previous
import functools

import jax
import jax.numpy as jnp
from jax import lax
from jax.experimental import pallas as pl
from jax.experimental.pallas import tpu as pltpu
from jax.experimental.pallas import tpu_sc as plsc

_H = 32
_W = 32
_D = 768
_NC = 2
_NS = 16
_NW = _NC * _NS
_L = 16
_WT = _W // 8
_DT = _D // 128

_mesh = plsc.VectorSubcoreMesh(core_axis_name="c", subcore_axis_name="s")


@functools.partial(
    pl.kernel,
    mesh=_mesh,
    out_type=jax.ShapeDtypeStruct((_H, _WT, _DT, 8, 128), jnp.float32),
    compiler_params=pltpu.CompilerParams(
        use_tc_tiling_on_sc=False, needs_layout_passes=False
    ),
    scratch_types=[
        pltpu.VMEM((_WT, _DT, 8, 128), jnp.float32),
        pltpu.VMEM((_WT, _DT, 8, 128), jnp.float32),
    ],
)
def _pos2d(row_hbm, col_hbm, out_hbm, buf_v, row_v):
    h = lax.axis_index("s") * _NC + lax.axis_index("c")
    ht = h // 8
    hs = h % 8
    pltpu.sync_copy(col_hbm, buf_v)
    pltpu.sync_copy(row_hbm, row_v)

    for dt in range(_DT):
        rvec = [row_v[ht, dt, hs, pl.ds(_L * k, _L)] for k in range(8)]
        for wt in range(_WT):
            for ws in range(8):
                for k in range(8):
                    plsc.addupdate(
                        buf_v.at[wt, dt, ws, pl.ds(_L * k, _L)], rvec[k]
                    )

    pltpu.sync_copy(buf_v, out_hbm.at[h])


def kernel(row_embed, col_embed):
    row4 = row_embed.reshape(_WT, 8, _DT, 128).transpose(0, 2, 1, 3)
    col4 = col_embed.reshape(_WT, 8, _DT, 128).transpose(0, 2, 1, 3)
    out5 = _pos2d(row4, col4)
    p = out5.transpose(0, 1, 3, 2, 4).reshape(_H, _W, _D)
    return jnp.transpose(p, (2, 0, 1))


# speedup vs baseline: 1.3988x; 1.0780x over previous
import functools

import jax
import jax.numpy as jnp
from jax import lax
from jax.experimental import pallas as pl
from jax.experimental.pallas import tpu as pltpu
from jax.experimental.pallas import tpu_sc as plsc

_H = 32
_W = 32
_D = 768
_NC = 2
_NS = 16
_NW = _NC * _NS
_L = 16
_WT = _W // 8
_DT = _D // 128

_mesh = plsc.VectorSubcoreMesh(core_axis_name="c", subcore_axis_name="s")


@functools.partial(
    pl.kernel,
    mesh=_mesh,
    out_type=jax.ShapeDtypeStruct((_H, _WT, _DT, 8, 128), jnp.float32),
    compiler_params=pltpu.CompilerParams(
        use_tc_tiling_on_sc=False, needs_layout_passes=False
    ),
    scratch_types=[
        pltpu.VMEM((_WT, _DT, 8, 128), jnp.float32),
        pltpu.VMEM((_DT, 128), jnp.float32),
        pltpu.SemaphoreType.DMA,
        [pltpu.SemaphoreType.DMA] * _WT,
        pltpu.SemaphoreType.DMA,
    ],
)
def _pos2d(row_hbm, col_hbm, out_hbm, buf_v, row_v, rsem, csems, osem):
    h = lax.axis_index("s") * _NC + lax.axis_index("c")
    ht = h // 8
    hs = h % 8

    rcopy = pltpu.make_async_copy(row_hbm.at[ht, :, hs], row_v, rsem)
    rcopy.start()
    ccopies = []
    for wt in range(_WT):
        c = pltpu.make_async_copy(col_hbm.at[wt], buf_v.at[wt], csems[wt])
        c.start()
        ccopies.append(c)
    rcopy.wait()
    rvec = [
        [row_v[dt, pl.ds(_L * k, _L)] for k in range(8)] for dt in range(_DT)
    ]

    ocopies = []
    for wt in range(_WT):
        ccopies[wt].wait()
        for dt in range(_DT):
            for ws in range(8):
                for k in range(8):
                    plsc.addupdate(
                        buf_v.at[wt, dt, ws, pl.ds(_L * k, _L)], rvec[dt][k]
                    )
        o = pltpu.make_async_copy(buf_v.at[wt], out_hbm.at[h, wt], osem)
        o.start()
        ocopies.append(o)
    for o in ocopies:
        o.wait()


def kernel(row_embed, col_embed):
    row4 = row_embed.reshape(_WT, 8, _DT, 128).transpose(0, 2, 1, 3)
    col4 = col_embed.reshape(_WT, 8, _DT, 128).transpose(0, 2, 1, 3)
    out5 = _pos2d(row4, col4)
    p = out5.transpose(0, 1, 3, 2, 4).reshape(_H, _W, _D)
    return jnp.transpose(p, (2, 0, 1))
